# baseline (speedup 1.0000x reference)
import functools

import jax
import jax.numpy as jnp
from jax import lax
from jax.experimental import pallas as pl
from jax.experimental.pallas import tpu as pltpu
from jax.experimental.pallas import tpu_sc as plsc

N = 50000
E = 800000
ORIG = 128
C = 64
DE = 16
H = 4
DH = C // H
NG = 256
HFE = 128
NCONV = 3

NPAD = 50176
ZROWS = NPAD // 16
NW = 32
ER = E // 100
BR = 5
RPW = ER // NW
RPS = ER // 16
RPD = ER // 32

_MESH = plsc.VectorSubcoreMesh(core_axis_name="c", subcore_axis_name="s")
_SC_PARAMS = pltpu.CompilerParams(needs_layout_passes=False,
                                  use_tc_tiling_on_sc=False)


def _emb_body(x_ref, w_ref, b_ref, o_ref):
    o_ref[...] = jnp.dot(x_ref[...], w_ref[...],
                         preferred_element_type=jnp.float32) + b_ref[...]


def _emb_matmul(x, W, b):
    n = x.shape[0]
    blk = 512
    return pl.pallas_call(
        _emb_body,
        grid=(n // blk,),
        in_specs=[
            pl.BlockSpec((blk, x.shape[1]), lambda i: (i, 0)),
            pl.BlockSpec((x.shape[1], W.shape[1]), lambda i: (0, 0)),
            pl.BlockSpec((1, W.shape[1]), lambda i: (0, 0)),
        ],
        out_specs=pl.BlockSpec((blk, W.shape[1]), lambda i: (i, 0)),
        out_shape=jax.ShapeDtypeStruct((n, W.shape[1]), jnp.float32),
    )(x, W, b.reshape(1, -1))


def _qkv_body(h_ref, wq_ref, bq_ref, wk_ref, bk_ref, wv_ref, bv_ref,
              q_ref, k_ref, v_ref):
    h = h_ref[...]
    q_ref[...] = jnp.dot(h, wq_ref[...], preferred_element_type=jnp.float32) + bq_ref[...]
    k_ref[...] = jnp.dot(h, wk_ref[...], preferred_element_type=jnp.float32) + bk_ref[...]
    v_ref[...] = jnp.dot(h, wv_ref[...], preferred_element_type=jnp.float32) + bv_ref[...]


def _qkv(h, Wq, bq, Wk, bk, Wv, bv):
    blk = 512
    full = lambda i: (0, 0)
    return pl.pallas_call(
        _qkv_body,
        grid=(NPAD // blk,),
        in_specs=[pl.BlockSpec((blk, C), lambda i: (i, 0))] +
                 [pl.BlockSpec((C, C), full), pl.BlockSpec((1, C), full)] * 3,
        out_specs=[pl.BlockSpec((blk, C), lambda i: (i, 0))] * 3,
        out_shape=[jax.ShapeDtypeStruct((NPAD, C), jnp.float32)] * 3,
    )(h, Wq, bq.reshape(1, -1), Wk, bk.reshape(1, -1), Wv, bv.reshape(1, -1))


def _efeat_body(ea_ref, we_ref, be_ref, e_ref):
    e_ref[...] = jnp.dot(ea_ref[...], we_ref[...],
                         preferred_element_type=jnp.float32) + be_ref[...]


def _efeat(edge_attr, We, be):
    blk = 1600
    return pl.pallas_call(
        _efeat_body,
        grid=(E // blk,),
        in_specs=[
            pl.BlockSpec((blk, DE), lambda i: (i, 0)),
            pl.BlockSpec((DE, C), lambda i: (0, 0)),
            pl.BlockSpec((1, C), lambda i: (0, 0)),
        ],
        out_specs=pl.BlockSpec((blk, C), lambda i: (i, 0)),
        out_shape=jax.ShapeDtypeStruct((E, C), jnp.float32),
    )(edge_attr, We, be.reshape(1, -1))


def _sc_gather(q, k, v, src2, dst2):
    @functools.partial(
        pl.kernel,
        out_type=[jax.ShapeDtypeStruct((ER, 100, 64), jnp.float32)] * 3,
        mesh=_MESH,
        compiler_params=_SC_PARAMS,
        scratch_types=[
            pltpu.VMEM((BR, 100), jnp.int32),
            pltpu.VMEM((BR, 100), jnp.int32),
            pltpu.VMEM((BR, 100, 64), jnp.float32),
            pltpu.VMEM((BR, 100, 64), jnp.float32),
            pltpu.VMEM((BR, 100, 64), jnp.float32),
            pltpu.SemaphoreType.DMA,
            pltpu.SemaphoreType.DMA,
            pltpu.SemaphoreType.DMA,
        ],
    )
    def kA(q_hbm, k_hbm, v_hbm, src_hbm, dst_hbm, qe_out, ks_out, vs_out,
           srcv, dstv, qb, kb, vb, semi, semg, sems):
        cid = lax.axis_index("c")
        sid = lax.axis_index("s")
        wid = sid * 2 + cid
        base = wid * RPW

        def chunk(c, _):
            row = base + c * BR
            ci = [pltpu.async_copy(src_hbm.at[pl.ds(row, BR)], srcv, semi),
                  pltpu.async_copy(dst_hbm.at[pl.ds(row, BR)], dstv, semi)]
            for cp in ci:
                cp.wait()
            cps = []
            for a in range(BR):
                cps.append(pltpu.async_copy(q_hbm.at[dstv.at[a]], qb.at[a], semg))
                cps.append(pltpu.async_copy(k_hbm.at[srcv.at[a]], kb.at[a], semg))
                cps.append(pltpu.async_copy(v_hbm.at[srcv.at[a]], vb.at[a], semg))
            for cp in cps:
                cp.wait()
            sts = []
            for a in range(BR):
                sts.append(pltpu.async_copy(qb.at[a], qe_out.at[row + a], sems))
                sts.append(pltpu.async_copy(kb.at[a], ks_out.at[row + a], sems))
                sts.append(pltpu.async_copy(vb.at[a], vs_out.at[row + a], sems))
            for st in sts:
                st.wait()
            return 0

        lax.fori_loop(0, RPW // BR, chunk, 0)

    return kA(q, k, v, src2, dst2)


def _edge_math_body(qe_ref, ks_ref, vs_ref, e_ref, ob_ref, rb_ref, eb_ref,
                    ex_ref, m0_ref, m1_ref):
    e = e_ref[...]
    kse = ks_ref[...] + e
    score = jnp.dot(qe_ref[...] * kse, ob_ref[...],
                    preferred_element_type=jnp.float32) * 0.25
    ex4 = jnp.exp(score)
    ex_ref[...] = jnp.dot(ex4, eb_ref[...], preferred_element_type=jnp.float32)
    msg = (vs_ref[...] + e) * jnp.dot(ex4, rb_ref[...],
                                      preferred_element_type=jnp.float32)
    m0_ref[...] = msg[:, :32]
    m1_ref[...] = msg[:, 32:]


def _edge_math(qe, ks, vs, e, OB, RB, EB):
    blk = 1600
    full = lambda i: (0, 0)
    return pl.pallas_call(
        _edge_math_body,
        grid=(E // blk,),
        in_specs=[pl.BlockSpec((blk, C), lambda i: (i, 0))] * 4 +
                 [pl.BlockSpec((C, 8), full), pl.BlockSpec((8, C), full),
                  pl.BlockSpec((8, 32), full)],
        out_specs=[
            pl.BlockSpec((blk, 32), lambda i: (i, 0)),
            pl.BlockSpec((blk, 32), lambda i: (i, 0)),
            pl.BlockSpec((blk, 32), lambda i: (i, 0)),
        ],
        out_shape=[
            jax.ShapeDtypeStruct((E, 32), jnp.float32),
            jax.ShapeDtypeStruct((E, 32), jnp.float32),
            jax.ShapeDtypeStruct((E, 32), jnp.float32),
        ],
    )(qe, ks, vs, e, OB, RB, EB)


def _sc_scatter(msg0, msg1, ex3, dst2, z32):
    @functools.partial(
        pl.kernel,
        out_type=[jax.ShapeDtypeStruct((NPAD, 32), jnp.float32),
                  jax.ShapeDtypeStruct((NPAD, 32), jnp.float32),
                  jax.ShapeDtypeStruct((2, NPAD, 32), jnp.float32)],
        mesh=_MESH,
        compiler_params=_SC_PARAMS,
        scratch_types=[
            pltpu.VMEM((BR, 100), jnp.int32),
            pltpu.VMEM((BR, 100, 32), jnp.float32),
            pltpu.VMEM_SHARED((NPAD, 32), jnp.float32),
            pltpu.SemaphoreType.DMA,
            pltpu.SemaphoreType.DMA,
            pltpu.SemaphoreType.DMA,
        ],
    )
    def kB(msg0_hbm, msg1_hbm, ex_hbm, dst_hbm, z_hbm,
           agg0_out, agg1_out, den_out,
           dstv0, mb0, acc_sh, semi, semm, sema):
        cid = lax.axis_index("c")
        sid = lax.axis_index("s")

        def sweep(val_hbm, base, nch):
            def chunk(c, _):
                row = base + c * BR
                loads = [pltpu.async_copy(dst_hbm.at[pl.ds(row, BR)],
                                          dstv0, semi),
                         pltpu.async_copy(val_hbm.at[pl.ds(row, BR)],
                                          mb0, semm)]
                for cp in loads:
                    cp.wait()
                adds = [pltpu.async_copy(mb0.at[a],
                                         acc_sh.at[dstv0.at[a]],
                                         sema, add=True)
                        for a in range(BR)]
                for cp in adds:
                    cp.wait()
                return 0

            lax.fori_loop(0, nch, chunk, 0)

        pltpu.sync_copy(z_hbm, acc_sh.at[pl.ds(sid * ZROWS, ZROWS)])
        plsc.subcore_barrier()

        @pl.when(cid == 0)
        def _():
            sweep(msg0_hbm, sid * RPS, RPS // BR)

        @pl.when(cid == 1)
        def _():
            sweep(msg1_hbm, sid * RPS, RPS // BR)

        plsc.subcore_barrier()

        @pl.when((sid == 0) & (cid == 0))
        def _():
            pltpu.sync_copy(acc_sh, agg0_out)

        @pl.when((sid == 0) & (cid == 1))
        def _():
            pltpu.sync_copy(acc_sh, agg1_out)

        plsc.subcore_barrier()

        pltpu.sync_copy(z_hbm, acc_sh.at[pl.ds(sid * ZROWS, ZROWS)])
        plsc.subcore_barrier()
        sweep(ex_hbm, cid * (ER // 2) + sid * RPD, RPD // BR)
        plsc.subcore_barrier()

        @pl.when(sid == 0)
        def _():
            pltpu.sync_copy(acc_sh, den_out.at[cid])

    return kB(msg0, msg1, ex3, dst2, z32)


def _postA_body(a0_ref, a1_ref, den_ref, rl_ref, rr_ref, wo_ref, bo_ref,
                t_ref, s_ref):
    i = pl.program_id(0)
    den = den_ref[0] + den_ref[1]
    dL = jnp.dot(den, rl_ref[...], preferred_element_type=jnp.float32) + 1e-16
    dR = jnp.dot(den, rr_ref[...], preferred_element_type=jnp.float32) + 1e-16
    t = (jnp.dot(a0_ref[...] / dL, wo_ref[0:32, :], preferred_element_type=jnp.float32)
         + jnp.dot(a1_ref[...] / dR, wo_ref[32:64, :], preferred_element_type=jnp.float32)
         + bo_ref[...])
    t_ref[...] = t
    rows = i * 512 + lax.broadcasted_iota(jnp.int32, (512, 1), 0)
    tm = jnp.where(rows < N, t, 0.0)
    part = jnp.concatenate([jnp.sum(tm, axis=0, keepdims=True),
                            jnp.sum(tm * tm, axis=0, keepdims=True),
                            jnp.zeros((6, C), jnp.float32)], axis=0)

    @pl.when(i == 0)
    def _():
        s_ref[...] = jnp.zeros_like(s_ref)

    s_ref[...] += part


def _postA(agg0, agg1, den, RL, RR, Wo, bo):
    blk = 512
    full = lambda i: (0, 0)
    return pl.pallas_call(
        _postA_body,
        grid=(NPAD // blk,),
        in_specs=[
            pl.BlockSpec((blk, 32), lambda i: (i, 0)),
            pl.BlockSpec((blk, 32), lambda i: (i, 0)),
            pl.BlockSpec((2, blk, 32), lambda i: (0, i, 0)),
            pl.BlockSpec((32, 32), full),
            pl.BlockSpec((32, 32), full),
            pl.BlockSpec((C, C), full),
            pl.BlockSpec((1, C), full),
        ],
        out_specs=[
            pl.BlockSpec((blk, C), lambda i: (i, 0)),
            pl.BlockSpec((8, C), full),
        ],
        out_shape=[
            jax.ShapeDtypeStruct((NPAD, C), jnp.float32),
            jax.ShapeDtypeStruct((8, C), jnp.float32),
        ],
    )(agg0, agg1, den, RL, RR, Wo, bo.reshape(1, -1))


def _postB_body(h_ref, t_ref, s_ref, g_ref, b_ref, o_ref):
    mu = s_ref[0:1, :] / N
    msq = s_ref[1:2, :] / N
    var = msq - mu * mu
    inv = lax.rsqrt(var + 1e-5)
    out = (t_ref[...] - mu) * inv * g_ref[...] + b_ref[...]
    o_ref[...] = jax.nn.softplus(h_ref[...] + out)


def _postB(h, t, sums, g, b):
    blk = 512
    full = lambda i: (0, 0)
    return pl.pallas_call(
        _postB_body,
        grid=(NPAD // blk,),
        in_specs=[
            pl.BlockSpec((blk, C), lambda i: (i, 0)),
            pl.BlockSpec((blk, C), lambda i: (i, 0)),
            pl.BlockSpec((8, C), full),
            pl.BlockSpec((1, C), full),
            pl.BlockSpec((1, C), full),
        ],
        out_specs=pl.BlockSpec((blk, C), lambda i: (i, 0)),
        out_shape=jax.ShapeDtypeStruct((NPAD, C), jnp.float32),
    )(h, t, sums, g.reshape(1, -1), b.reshape(1, -1))


def _pool_body(h_ref, b_ref, p_ref, c_ref):
    i = pl.program_id(0)
    gids = lax.broadcasted_iota(jnp.int32, (NG, 512), 0).astype(jnp.float32)
    onehotT = jnp.where(b_ref[...] == gids, 1.0, 0.0)
    part_p = jnp.dot(onehotT, h_ref[...], preferred_element_type=jnp.float32)
    part_c = jnp.sum(onehotT, axis=1, keepdims=True)

    @pl.when(i == 0)
    def _():
        p_ref[...] = jnp.zeros_like(p_ref)
        c_ref[...] = jnp.zeros_like(c_ref)

    p_ref[...] += part_p
    c_ref[...] += part_c * jnp.ones((1, 8), jnp.float32)


def _pool(h, batchf):
    blk = 512
    return pl.pallas_call(
        _pool_body,
        grid=(NPAD // blk,),
        in_specs=[
            pl.BlockSpec((blk, C), lambda i: (i, 0)),
            pl.BlockSpec((1, blk), lambda i: (0, i)),
        ],
        out_specs=[
            pl.BlockSpec((NG, C), lambda i: (0, 0)),
            pl.BlockSpec((NG, 8), lambda i: (0, 0)),
        ],
        out_shape=[
            jax.ShapeDtypeStruct((NG, C), jnp.float32),
            jax.ShapeDtypeStruct((NG, 8), jnp.float32),
        ],
    )(h, batchf)


def _fc_body(p_ref, c_ref, wfc_ref, bfc_ref, wout_ref, bout_ref, o_ref):
    cnt = jnp.maximum(c_ref[:, 0:1], 1.0)
    p = jax.nn.softplus(p_ref[...] / cnt)
    t = jnp.dot(p, wfc_ref[...], preferred_element_type=jnp.float32) + bfc_ref[...]
    t = jax.nn.softplus(t)
    o_ref[...] = jnp.dot(t, wout_ref[...], preferred_element_type=jnp.float32) + bout_ref[...]


def _fc_head(pooled, counts, W_fc, b_fc, W_out, b_out):
    return pl.pallas_call(
        _fc_body,
        out_shape=jax.ShapeDtypeStruct((NG, 1), jnp.float32),
    )(pooled, counts, W_fc, b_fc.reshape(1, -1), W_out, b_out.reshape(1, -1))


def kernel(x, edge_index, edge_attr, batch, W_emb, b_emb, Wq, bq, Wk, bk,
           Wv, bv, We, be, Wo, bo, bn_g, bn_b, W_fc, b_fc, W_out, b_out):
    src2 = edge_index[0].reshape(ER, 100)
    dst2 = edge_index[1].reshape(ER, 100)
    z32 = jnp.zeros((ZROWS, 32), jnp.float32)
    hsel = (jnp.arange(64)[:, None] // 16 == jnp.arange(4)[None, :]).astype(jnp.float32)
    OB = jnp.concatenate([hsel, jnp.zeros((64, 4), jnp.float32)], axis=1)
    RB = jnp.concatenate([hsel.T, jnp.zeros((4, 64), jnp.float32)], axis=0)
    EB = jnp.concatenate([jnp.eye(4, dtype=jnp.float32),
                          jnp.zeros((4, 28), jnp.float32)], axis=1)
    EB = jnp.concatenate([EB, jnp.zeros((4, 32), jnp.float32)], axis=0)
    eye4 = jnp.eye(4, dtype=jnp.float32)
    RL = jnp.concatenate(
        [jnp.repeat(eye4[:, 0:2], 16, axis=1), jnp.zeros((28, 32), jnp.float32)], axis=0)
    RR = jnp.concatenate(
        [jnp.repeat(eye4[:, 2:4], 16, axis=1), jnp.zeros((28, 32), jnp.float32)], axis=0)
    batchf = jnp.concatenate(
        [batch.astype(jnp.float32), jnp.full((NPAD - N,), 2.0 * NG, jnp.float32)]
    ).reshape(1, NPAD)

    xp = jnp.concatenate([x, jnp.zeros((NPAD - N, ORIG), jnp.float32)], axis=0)
    h = _emb_matmul(xp, W_emb, b_emb)

    for i in range(NCONV):
        q, k, v = _qkv(h, Wq[i], bq[i], Wk[i], bk[i], Wv[i], bv[i])
        e = _efeat(edge_attr, We[i], be[i])
        qe, ks, vs = _sc_gather(q, k, v, src2, dst2)
        ex, msg0, msg1 = _edge_math(qe.reshape(E, 64), ks.reshape(E, 64),
                                    vs.reshape(E, 64), e, OB, RB, EB)
        agg0, agg1, den = _sc_scatter(msg0.reshape(ER, 100, 32),
                                      msg1.reshape(ER, 100, 32),
                                      ex.reshape(ER, 100, 32), dst2, z32)
        t, sums = _postA(agg0, agg1, den, RL, RR, Wo[i], bo[i])
        h = _postB(h, t, sums, bn_g[i], bn_b[i])

    pooled, counts = _pool(h, batchf)
    return _fc_head(pooled, counts, W_fc, b_fc, W_out, b_out)


# speedup vs baseline: 1.0698x; 1.0698x over previous
import functools

import jax
import jax.numpy as jnp
from jax import lax
from jax.experimental import pallas as pl
from jax.experimental.pallas import tpu as pltpu
from jax.experimental.pallas import tpu_sc as plsc

N = 50000
E = 800000
ORIG = 128
C = 64
DE = 16
H = 4
DH = C // H
NG = 256
HFE = 128
NCONV = 3

NPAD = 50176
ZROWS = NPAD // 16
NW = 32
ER = E // 100
BR = 5
RPW = ER // NW
RPS = ER // 16
RPD = ER // 32

_MESH = plsc.VectorSubcoreMesh(core_axis_name="c", subcore_axis_name="s")
_SC_PARAMS = pltpu.CompilerParams(needs_layout_passes=False,
                                  use_tc_tiling_on_sc=False)


def _emb_body(x_ref, w_ref, b_ref, o_ref):
    o_ref[...] = jnp.dot(x_ref[...], w_ref[...],
                         preferred_element_type=jnp.float32) + b_ref[...]


def _emb_matmul(x, W, b):
    n = x.shape[0]
    blk = 512
    return pl.pallas_call(
        _emb_body,
        grid=(n // blk,),
        in_specs=[
            pl.BlockSpec((blk, x.shape[1]), lambda i: (i, 0)),
            pl.BlockSpec((x.shape[1], W.shape[1]), lambda i: (0, 0)),
            pl.BlockSpec((1, W.shape[1]), lambda i: (0, 0)),
        ],
        out_specs=pl.BlockSpec((blk, W.shape[1]), lambda i: (i, 0)),
        out_shape=jax.ShapeDtypeStruct((n, W.shape[1]), jnp.float32),
    )(x, W, b.reshape(1, -1))


def _qkv_body(h_ref, wq_ref, bq_ref, wk_ref, bk_ref, wv_ref, bv_ref,
              q_ref, k_ref, v_ref):
    h = h_ref[...]
    q_ref[...] = jnp.dot(h, wq_ref[...], preferred_element_type=jnp.float32) + bq_ref[...]
    k_ref[...] = jnp.dot(h, wk_ref[...], preferred_element_type=jnp.float32) + bk_ref[...]
    v_ref[...] = jnp.dot(h, wv_ref[...], preferred_element_type=jnp.float32) + bv_ref[...]


def _qkv(h, Wq, bq, Wk, bk, Wv, bv):
    blk = 512
    full = lambda i: (0, 0)
    return pl.pallas_call(
        _qkv_body,
        grid=(NPAD // blk,),
        in_specs=[pl.BlockSpec((blk, C), lambda i: (i, 0))] +
                 [pl.BlockSpec((C, C), full), pl.BlockSpec((1, C), full)] * 3,
        out_specs=[pl.BlockSpec((blk, C), lambda i: (i, 0))] * 3,
        out_shape=[jax.ShapeDtypeStruct((NPAD, C), jnp.float32)] * 3,
    )(h, Wq, bq.reshape(1, -1), Wk, bk.reshape(1, -1), Wv, bv.reshape(1, -1))


def _efeat_body(ea_ref, we_ref, be_ref, e_ref):
    e_ref[...] = jnp.dot(ea_ref[...], we_ref[...],
                         preferred_element_type=jnp.float32) + be_ref[...]


def _efeat(edge_attr, We, be):
    blk = 1600
    return pl.pallas_call(
        _efeat_body,
        grid=(E // blk,),
        in_specs=[
            pl.BlockSpec((blk, DE), lambda i: (i, 0)),
            pl.BlockSpec((DE, C), lambda i: (0, 0)),
            pl.BlockSpec((1, C), lambda i: (0, 0)),
        ],
        out_specs=pl.BlockSpec((blk, C), lambda i: (i, 0)),
        out_shape=jax.ShapeDtypeStruct((E, C), jnp.float32),
    )(edge_attr, We, be.reshape(1, -1))


def _sc_gather(q, k, v, src2, dst2):
    @functools.partial(
        pl.kernel,
        out_type=[jax.ShapeDtypeStruct((ER, 100, 64), jnp.float32)] * 3,
        mesh=_MESH,
        compiler_params=_SC_PARAMS,
        scratch_types=[
            pltpu.VMEM((BR, 100), jnp.int32),
            pltpu.VMEM((BR, 100), jnp.int32),
            pltpu.VMEM((BR, 100, 64), jnp.float32),
            pltpu.VMEM((BR, 100, 64), jnp.float32),
            pltpu.VMEM((BR, 100, 64), jnp.float32),
            pltpu.SemaphoreType.DMA,
            pltpu.SemaphoreType.DMA,
            pltpu.SemaphoreType.DMA,
        ],
    )
    def kA(q_hbm, k_hbm, v_hbm, src_hbm, dst_hbm, qe_out, ks_out, vs_out,
           srcv, dstv, qb, kb, vb, semi, semg, sems):
        cid = lax.axis_index("c")
        sid = lax.axis_index("s")
        wid = sid * 2 + cid
        base = wid * RPW

        def chunk(c, _):
            row = base + c * BR
            ci = [pltpu.async_copy(src_hbm.at[pl.ds(row, BR)], srcv, semi),
                  pltpu.async_copy(dst_hbm.at[pl.ds(row, BR)], dstv, semi)]
            for cp in ci:
                cp.wait()
            cps = []
            for a in range(BR):
                cps.append(pltpu.async_copy(q_hbm.at[dstv.at[a]], qb.at[a], semg))
                cps.append(pltpu.async_copy(k_hbm.at[srcv.at[a]], kb.at[a], semg))
                cps.append(pltpu.async_copy(v_hbm.at[srcv.at[a]], vb.at[a], semg))
            for cp in cps:
                cp.wait()
            sts = []
            for a in range(BR):
                sts.append(pltpu.async_copy(qb.at[a], qe_out.at[row + a], sems))
                sts.append(pltpu.async_copy(kb.at[a], ks_out.at[row + a], sems))
                sts.append(pltpu.async_copy(vb.at[a], vs_out.at[row + a], sems))
            for st in sts:
                st.wait()
            return 0

        lax.fori_loop(0, RPW // BR, chunk, 0)

    return kA(q, k, v, src2, dst2)


def _edge_math_body(qe_ref, ks_ref, vs_ref, e_ref, ob_ref, rb_ref, eb_ref,
                    ex_ref, m0_ref, m1_ref):
    e = e_ref[...]
    kse = ks_ref[...] + e
    score = jnp.dot(qe_ref[...] * kse, ob_ref[...],
                    preferred_element_type=jnp.float32) * 0.25
    ex4 = jnp.exp(score)
    ex_ref[...] = jnp.dot(ex4, eb_ref[...], preferred_element_type=jnp.float32)
    msg = (vs_ref[...] + e) * jnp.dot(ex4, rb_ref[...],
                                      preferred_element_type=jnp.float32)
    m0_ref[...] = msg[:, :32]
    m1_ref[...] = msg[:, 32:]


def _edge_math(qe, ks, vs, e, OB, RB, EB):
    blk = 1600
    full = lambda i: (0, 0)
    return pl.pallas_call(
        _edge_math_body,
        grid=(E // blk,),
        in_specs=[pl.BlockSpec((blk, C), lambda i: (i, 0))] * 4 +
                 [pl.BlockSpec((C, 8), full), pl.BlockSpec((8, C), full),
                  pl.BlockSpec((8, 16), full)],
        out_specs=[
            pl.BlockSpec((blk, 16), lambda i: (i, 0)),
            pl.BlockSpec((blk, 32), lambda i: (i, 0)),
            pl.BlockSpec((blk, 32), lambda i: (i, 0)),
        ],
        out_shape=[
            jax.ShapeDtypeStruct((E, 16), jnp.float32),
            jax.ShapeDtypeStruct((E, 32), jnp.float32),
            jax.ShapeDtypeStruct((E, 32), jnp.float32),
        ],
    )(qe, ks, vs, e, OB, RB, EB)


def _sc_scatter_agg(msg0, msg1, dst2, z32):
    @functools.partial(
        pl.kernel,
        out_type=[jax.ShapeDtypeStruct((NPAD, 32), jnp.float32)] * 2,
        mesh=_MESH,
        compiler_params=_SC_PARAMS,
        scratch_types=[
            pltpu.VMEM((BR, 100), jnp.int32),
            pltpu.VMEM((BR, 100, 32), jnp.float32),
            pltpu.VMEM_SHARED((NPAD, 32), jnp.float32),
            pltpu.SemaphoreType.DMA,
            pltpu.SemaphoreType.DMA,
            pltpu.SemaphoreType.DMA,
        ],
    )
    def kB(msg0_hbm, msg1_hbm, dst_hbm, z_hbm, agg0_out, agg1_out,
           dstv, mb, agg_sh, semi, semm, sema):
        cid = lax.axis_index("c")
        sid = lax.axis_index("s")
        pltpu.sync_copy(z_hbm, agg_sh.at[pl.ds(sid * ZROWS, ZROWS)])
        plsc.subcore_barrier()
        base = sid * RPS

        def make_chunk(msg_hbm):
            def chunk(c, _):
                row = base + c * BR
                ci = [pltpu.async_copy(dst_hbm.at[pl.ds(row, BR)], dstv, semi),
                      pltpu.async_copy(msg_hbm.at[pl.ds(row, BR)], mb, semm)]
                for cp in ci:
                    cp.wait()
                adds = [pltpu.async_copy(mb.at[a], agg_sh.at[dstv.at[a]],
                                         sema, add=True) for a in range(BR)]
                for cp in adds:
                    cp.wait()
                return 0
            return chunk

        @pl.when(cid == 0)
        def _():
            lax.fori_loop(0, RPS // BR, make_chunk(msg0_hbm), 0)

        @pl.when(cid == 1)
        def _():
            lax.fori_loop(0, RPS // BR, make_chunk(msg1_hbm), 0)

        plsc.subcore_barrier()

        @pl.when((sid == 0) & (cid == 0))
        def _():
            pltpu.sync_copy(agg_sh, agg0_out)

        @pl.when((sid == 0) & (cid == 1))
        def _():
            pltpu.sync_copy(agg_sh, agg1_out)

    return kB(msg0, msg1, dst2, z32)


def _sc_scatter_den(ex3, dst2, z16):
    @functools.partial(
        pl.kernel,
        out_type=jax.ShapeDtypeStruct((2, NPAD, 16), jnp.float32),
        mesh=_MESH,
        compiler_params=_SC_PARAMS,
        scratch_types=[
            pltpu.VMEM((BR, 100), jnp.int32),
            pltpu.VMEM((BR, 100, 16), jnp.float32),
            pltpu.VMEM_SHARED((NPAD, 16), jnp.float32),
            pltpu.SemaphoreType.DMA,
            pltpu.SemaphoreType.DMA,
            pltpu.SemaphoreType.DMA,
        ],
    )
    def kC(ex_hbm, dst_hbm, z_hbm, den_out, dstv, eb, den_sh, semi, seme,
           sema):
        cid = lax.axis_index("c")
        sid = lax.axis_index("s")
        pltpu.sync_copy(z_hbm, den_sh.at[pl.ds(sid * ZROWS, ZROWS)])
        plsc.subcore_barrier()
        base = cid * (ER // 2) + sid * RPD

        def chunk(c, _):
            row = base + c * BR
            ci = [pltpu.async_copy(dst_hbm.at[pl.ds(row, BR)], dstv, semi),
                  pltpu.async_copy(ex_hbm.at[pl.ds(row, BR)], eb, seme)]
            for cp in ci:
                cp.wait()
            adds = [pltpu.async_copy(eb.at[a], den_sh.at[dstv.at[a]],
                                     sema, add=True) for a in range(BR)]
            for cp in adds:
                cp.wait()
            return 0

        lax.fori_loop(0, RPD // BR, chunk, 0)
        plsc.subcore_barrier()

        @pl.when(sid == 0)
        def _():
            pltpu.sync_copy(den_sh, den_out.at[cid])

    return kC(ex3, dst2, z16)


def _postA_body(a0_ref, a1_ref, den_ref, rl_ref, rr_ref, wo_ref, bo_ref,
                t_ref, s_ref):
    i = pl.program_id(0)
    den = den_ref[0] + den_ref[1]
    dL = jnp.dot(den, rl_ref[...], preferred_element_type=jnp.float32) + 1e-16
    dR = jnp.dot(den, rr_ref[...], preferred_element_type=jnp.float32) + 1e-16
    t = (jnp.dot(a0_ref[...] / dL, wo_ref[0:32, :], preferred_element_type=jnp.float32)
         + jnp.dot(a1_ref[...] / dR, wo_ref[32:64, :], preferred_element_type=jnp.float32)
         + bo_ref[...])
    t_ref[...] = t
    rows = i * 512 + lax.broadcasted_iota(jnp.int32, (512, 1), 0)
    tm = jnp.where(rows < N, t, 0.0)
    part = jnp.concatenate([jnp.sum(tm, axis=0, keepdims=True),
                            jnp.sum(tm * tm, axis=0, keepdims=True),
                            jnp.zeros((6, C), jnp.float32)], axis=0)

    @pl.when(i == 0)
    def _():
        s_ref[...] = jnp.zeros_like(s_ref)

    s_ref[...] += part


def _postA(agg0, agg1, den, RL, RR, Wo, bo):
    blk = 512
    full = lambda i: (0, 0)
    return pl.pallas_call(
        _postA_body,
        grid=(NPAD // blk,),
        in_specs=[
            pl.BlockSpec((blk, 32), lambda i: (i, 0)),
            pl.BlockSpec((blk, 32), lambda i: (i, 0)),
            pl.BlockSpec((2, blk, 16), lambda i: (0, i, 0)),
            pl.BlockSpec((16, 32), full),
            pl.BlockSpec((16, 32), full),
            pl.BlockSpec((C, C), full),
            pl.BlockSpec((1, C), full),
        ],
        out_specs=[
            pl.BlockSpec((blk, C), lambda i: (i, 0)),
            pl.BlockSpec((8, C), full),
        ],
        out_shape=[
            jax.ShapeDtypeStruct((NPAD, C), jnp.float32),
            jax.ShapeDtypeStruct((8, C), jnp.float32),
        ],
    )(agg0, agg1, den, RL, RR, Wo, bo.reshape(1, -1))


def _postB_body(h_ref, t_ref, s_ref, g_ref, b_ref, o_ref):
    mu = s_ref[0:1, :] / N
    msq = s_ref[1:2, :] / N
    var = msq - mu * mu
    inv = lax.rsqrt(var + 1e-5)
    out = (t_ref[...] - mu) * inv * g_ref[...] + b_ref[...]
    o_ref[...] = jax.nn.softplus(h_ref[...] + out)


def _postB(h, t, sums, g, b):
    blk = 512
    full = lambda i: (0, 0)
    return pl.pallas_call(
        _postB_body,
        grid=(NPAD // blk,),
        in_specs=[
            pl.BlockSpec((blk, C), lambda i: (i, 0)),
            pl.BlockSpec((blk, C), lambda i: (i, 0)),
            pl.BlockSpec((8, C), full),
            pl.BlockSpec((1, C), full),
            pl.BlockSpec((1, C), full),
        ],
        out_specs=pl.BlockSpec((blk, C), lambda i: (i, 0)),
        out_shape=jax.ShapeDtypeStruct((NPAD, C), jnp.float32),
    )(h, t, sums, g.reshape(1, -1), b.reshape(1, -1))


def _pool_body(h_ref, b_ref, p_ref, c_ref):
    i = pl.program_id(0)
    gids = lax.broadcasted_iota(jnp.int32, (NG, 512), 0).astype(jnp.float32)
    onehotT = jnp.where(b_ref[...] == gids, 1.0, 0.0)
    part_p = jnp.dot(onehotT, h_ref[...], preferred_element_type=jnp.float32)
    part_c = jnp.sum(onehotT, axis=1, keepdims=True)

    @pl.when(i == 0)
    def _():
        p_ref[...] = jnp.zeros_like(p_ref)
        c_ref[...] = jnp.zeros_like(c_ref)

    p_ref[...] += part_p
    c_ref[...] += part_c * jnp.ones((1, 8), jnp.float32)


def _pool(h, batchf):
    blk = 512
    return pl.pallas_call(
        _pool_body,
        grid=(NPAD // blk,),
        in_specs=[
            pl.BlockSpec((blk, C), lambda i: (i, 0)),
            pl.BlockSpec((1, blk), lambda i: (0, i)),
        ],
        out_specs=[
            pl.BlockSpec((NG, C), lambda i: (0, 0)),
            pl.BlockSpec((NG, 8), lambda i: (0, 0)),
        ],
        out_shape=[
            jax.ShapeDtypeStruct((NG, C), jnp.float32),
            jax.ShapeDtypeStruct((NG, 8), jnp.float32),
        ],
    )(h, batchf)


def _fc_body(p_ref, c_ref, wfc_ref, bfc_ref, wout_ref, bout_ref, o_ref):
    cnt = jnp.maximum(c_ref[:, 0:1], 1.0)
    p = jax.nn.softplus(p_ref[...] / cnt)
    t = jnp.dot(p, wfc_ref[...], preferred_element_type=jnp.float32) + bfc_ref[...]
    t = jax.nn.softplus(t)
    o_ref[...] = jnp.dot(t, wout_ref[...], preferred_element_type=jnp.float32) + bout_ref[...]


def _fc_head(pooled, counts, W_fc, b_fc, W_out, b_out):
    return pl.pallas_call(
        _fc_body,
        out_shape=jax.ShapeDtypeStruct((NG, 1), jnp.float32),
    )(pooled, counts, W_fc, b_fc.reshape(1, -1), W_out, b_out.reshape(1, -1))


def kernel(x, edge_index, edge_attr, batch, W_emb, b_emb, Wq, bq, Wk, bk,
           Wv, bv, We, be, Wo, bo, bn_g, bn_b, W_fc, b_fc, W_out, b_out):
    src2 = edge_index[0].reshape(ER, 100)
    dst2 = edge_index[1].reshape(ER, 100)
    z16 = jnp.zeros((ZROWS, 16), jnp.float32)
    z32 = jnp.zeros((ZROWS, 32), jnp.float32)
    hsel = (jnp.arange(64)[:, None] // 16 == jnp.arange(4)[None, :]).astype(jnp.float32)
    OB = jnp.concatenate([hsel, jnp.zeros((64, 4), jnp.float32)], axis=1)
    RB = jnp.concatenate([hsel.T, jnp.zeros((4, 64), jnp.float32)], axis=0)
    EB = jnp.concatenate([jnp.eye(4, dtype=jnp.float32),
                          jnp.zeros((4, 12), jnp.float32)], axis=1)
    EB = jnp.concatenate([EB, jnp.zeros((4, 16), jnp.float32)], axis=0)
    eye4 = jnp.eye(4, dtype=jnp.float32)
    RL = jnp.concatenate(
        [jnp.repeat(eye4[:, 0:2], 16, axis=1), jnp.zeros((12, 32), jnp.float32)], axis=0)
    RR = jnp.concatenate(
        [jnp.repeat(eye4[:, 2:4], 16, axis=1), jnp.zeros((12, 32), jnp.float32)], axis=0)
    batchf = jnp.concatenate(
        [batch.astype(jnp.float32), jnp.full((NPAD - N,), 2.0 * NG, jnp.float32)]
    ).reshape(1, NPAD)

    xp = jnp.concatenate([x, jnp.zeros((NPAD - N, ORIG), jnp.float32)], axis=0)
    h = _emb_matmul(xp, W_emb, b_emb)

    for i in range(NCONV):
        q, k, v = _qkv(h, Wq[i], bq[i], Wk[i], bk[i], Wv[i], bv[i])
        e = _efeat(edge_attr, We[i], be[i])
        qe, ks, vs = _sc_gather(q, k, v, src2, dst2)
        ex, msg0, msg1 = _edge_math(qe.reshape(E, 64), ks.reshape(E, 64),
                                    vs.reshape(E, 64), e, OB, RB, EB)
        agg0, agg1 = _sc_scatter_agg(msg0.reshape(ER, 100, 32),
                                     msg1.reshape(ER, 100, 32), dst2, z32)
        den = _sc_scatter_den(ex.reshape(ER, 100, 16), dst2, z16)
        t, sums = _postA(agg0, agg1, den, RL, RR, Wo[i], bo[i])
        h = _postB(h, t, sums, bn_g[i], bn_b[i])

    pooled, counts = _pool(h, batchf)
    return _fc_head(pooled, counts, W_fc, b_fc, W_out, b_out)
